# baseline (device time: 409683 ns/iter reference)
import jax
import jax.numpy as jnp
from jax import lax
from jax.experimental import pallas as pl
from jax.experimental.pallas import tpu as pltpu

N_DEV = 16
M = 2048
CHUNK = M // N_DEV

H_RING = [0, 1, 5, 9, 13, 14, 10, 6, 2, 3, 7, 11, 15, 12, 8, 4]
POS = [0] * N_DEV
NEXT = [0] * N_DEV
PREV = [0] * N_DEV
for _p, _l in enumerate(H_RING):
    POS[_l] = _p
    NEXT[_l] = H_RING[(_p + 1) % N_DEV]
    PREV[_l] = H_RING[(_p - 1) % N_DEV]


def kernel(x, w_mat):
    my = lax.axis_index("i")
    meta = jnp.stack(
        [
            jnp.asarray(POS, jnp.int32)[my],
            jnp.asarray(NEXT, jnp.int32)[my],
            jnp.asarray(PREV, jnp.int32)[my],
        ]
    )

    def body(x_ref, w_ref, meta_ref, out_ref, comm, rs_send, rs_recv,
             ag_send, ag_recv):
        pos = meta_ref[0]
        right = meta_ref[1]
        left = meta_ref[2]

        barrier_sem = pltpu.get_barrier_semaphore()
        pl.semaphore_signal(barrier_sem, 1, device_id=(right,),
                            device_id_type=pl.DeviceIdType.MESH)
        pl.semaphore_signal(barrier_sem, 1, device_id=(left,),
                            device_id_type=pl.DeviceIdType.MESH)
        pl.semaphore_wait(barrier_sem, 2)

        def local_chunk(c):
            return jnp.dot(
                x_ref[pl.ds(c * CHUNK, CHUNK), :],
                w_ref[:, :],
                preferred_element_type=jnp.float32,
            )

        comm[0, :, :] = local_chunk(pos)
        for s in range(N_DEV - 1):
            rdma = pltpu.make_async_remote_copy(
                src_ref=comm.at[s],
                dst_ref=comm.at[s + 1],
                send_sem=rs_send.at[s],
                recv_sem=rs_recv.at[s],
                device_id=(right,),
                device_id_type=pl.DeviceIdType.MESH,
            )
            rdma.start()
            rdma.wait()
            c_recv = (pos - s - 1) % N_DEV
            comm[s + 1, :, :] = comm[s + 1, :, :] + local_chunk(c_recv)

        own = (pos + 1) % N_DEV
        out_ref[pl.ds(own * CHUNK, CHUNK), :] = jnp.maximum(
            comm[N_DEV - 1, :, :], 0.0
        )

        for s in range(N_DEV - 1):
            cs = (pos + 1 - s) % N_DEV
            rows = pl.ds(cs * CHUNK, CHUNK)
            rdma = pltpu.make_async_remote_copy(
                src_ref=out_ref.at[rows, :],
                dst_ref=out_ref.at[rows, :],
                send_sem=ag_send.at[s],
                recv_sem=ag_recv.at[s],
                device_id=(right,),
                device_id_type=pl.DeviceIdType.MESH,
            )
            rdma.start()
            rdma.wait()

    return pl.pallas_call(
        body,
        out_shape=jax.ShapeDtypeStruct((M, M), jnp.float32),
        in_specs=[
            pl.BlockSpec(memory_space=pltpu.VMEM),
            pl.BlockSpec(memory_space=pltpu.VMEM),
            pl.BlockSpec(memory_space=pltpu.SMEM),
        ],
        out_specs=pl.BlockSpec(memory_space=pltpu.VMEM),
        scratch_shapes=[
            pltpu.VMEM((N_DEV, CHUNK, M), jnp.float32),
            pltpu.SemaphoreType.DMA((N_DEV - 1,)),
            pltpu.SemaphoreType.DMA((N_DEV - 1,)),
            pltpu.SemaphoreType.DMA((N_DEV - 1,)),
            pltpu.SemaphoreType.DMA((N_DEV - 1,)),
        ],
        compiler_params=pltpu.CompilerParams(collective_id=0),
    )(x, w_mat, meta)


# device time: 240331 ns/iter; 1.7047x vs baseline; 1.7047x over previous
import jax
import jax.numpy as jnp
from jax import lax
from jax.experimental import pallas as pl
from jax.experimental.pallas import tpu as pltpu

N_DEV = 16
M = 2048
HALF = M // 2
CHUNK = HALF // N_DEV

H_RING = [0, 1, 5, 9, 13, 14, 10, 6, 2, 3, 7, 11, 15, 12, 8, 4]
POS = [0] * N_DEV
NEXT = [0] * N_DEV
PREV = [0] * N_DEV
for _p, _l in enumerate(H_RING):
    POS[_l] = _p
    NEXT[_l] = H_RING[(_p + 1) % N_DEV]
    PREV[_l] = H_RING[(_p - 1) % N_DEV]


def kernel(x, w_mat):
    my = lax.axis_index("i")
    meta = jnp.stack(
        [
            jnp.asarray(POS, jnp.int32)[my],
            jnp.asarray(NEXT, jnp.int32)[my],
            jnp.asarray(PREV, jnp.int32)[my],
        ]
    )

    def body(x_ref, w_ref, meta_ref, out_ref, comm_cw, comm_ccw,
             rs_send_cw, rs_recv_cw, rs_send_ccw, rs_recv_ccw,
             ag_send_cw, ag_recv_cw, ag_send_ccw, ag_recv_ccw):
        pos = meta_ref[0]
        right = meta_ref[1]
        left = meta_ref[2]
        qos = (N_DEV - pos) % N_DEV

        def local_chunk(c, base):
            return jnp.dot(
                x_ref[pl.ds(base + c * CHUNK, CHUNK), :],
                w_ref[:, :],
                preferred_element_type=jnp.float32,
            )

        comm_cw[0, :, :] = local_chunk(pos, 0)
        comm_ccw[0, :, :] = local_chunk(qos, HALF)

        barrier_sem = pltpu.get_barrier_semaphore()
        pl.semaphore_signal(barrier_sem, 1, device_id=(right,),
                            device_id_type=pl.DeviceIdType.MESH)
        pl.semaphore_signal(barrier_sem, 1, device_id=(left,),
                            device_id_type=pl.DeviceIdType.MESH)
        pl.semaphore_wait(barrier_sem, 2)

        pending = []

        for s in range(N_DEV - 1):
            r_cw = pltpu.make_async_remote_copy(
                src_ref=comm_cw.at[s],
                dst_ref=comm_cw.at[s + 1],
                send_sem=rs_send_cw.at[s],
                recv_sem=rs_recv_cw.at[s],
                device_id=(right,),
                device_id_type=pl.DeviceIdType.MESH,
            )
            r_ccw = pltpu.make_async_remote_copy(
                src_ref=comm_ccw.at[s],
                dst_ref=comm_ccw.at[s + 1],
                send_sem=rs_send_ccw.at[s],
                recv_sem=rs_recv_ccw.at[s],
                device_id=(left,),
                device_id_type=pl.DeviceIdType.MESH,
            )
            r_cw.start()
            r_ccw.start()
            a_cw = local_chunk((pos - s - 1) % N_DEV, 0)
            a_ccw = local_chunk((qos - s - 1) % N_DEV, HALF)
            r_cw.wait_recv()
            r_ccw.wait_recv()
            comm_cw[s + 1, :, :] = comm_cw[s + 1, :, :] + a_cw
            comm_ccw[s + 1, :, :] = comm_ccw[s + 1, :, :] + a_ccw
            pending.append(r_cw)
            pending.append(r_ccw)

        own_cw = (pos + 1) % N_DEV
        own_ccw = (qos + 1) % N_DEV
        out_ref[pl.ds(own_cw * CHUNK, CHUNK), :] = jnp.maximum(
            comm_cw[N_DEV - 1, :, :], 0.0
        )
        out_ref[pl.ds(HALF + own_ccw * CHUNK, CHUNK), :] = jnp.maximum(
            comm_ccw[N_DEV - 1, :, :], 0.0
        )

        for s in range(N_DEV - 1):
            cs_cw = (pos + 1 - s) % N_DEV
            cs_ccw = (qos + 1 - s) % N_DEV
            rows_cw = pl.ds(cs_cw * CHUNK, CHUNK)
            rows_ccw = pl.ds(HALF + cs_ccw * CHUNK, CHUNK)
            g_cw = pltpu.make_async_remote_copy(
                src_ref=out_ref.at[rows_cw, :],
                dst_ref=out_ref.at[rows_cw, :],
                send_sem=ag_send_cw.at[s],
                recv_sem=ag_recv_cw.at[s],
                device_id=(right,),
                device_id_type=pl.DeviceIdType.MESH,
            )
            g_ccw = pltpu.make_async_remote_copy(
                src_ref=out_ref.at[rows_ccw, :],
                dst_ref=out_ref.at[rows_ccw, :],
                send_sem=ag_send_ccw.at[s],
                recv_sem=ag_recv_ccw.at[s],
                device_id=(left,),
                device_id_type=pl.DeviceIdType.MESH,
            )
            g_cw.start()
            g_ccw.start()
            g_cw.wait_recv()
            g_ccw.wait_recv()
            pending.append(g_cw)
            pending.append(g_ccw)

        for r in pending:
            r.wait_send()

    return pl.pallas_call(
        body,
        out_shape=jax.ShapeDtypeStruct((M, M), jnp.float32),
        in_specs=[
            pl.BlockSpec(memory_space=pltpu.VMEM),
            pl.BlockSpec(memory_space=pltpu.VMEM),
            pl.BlockSpec(memory_space=pltpu.SMEM),
        ],
        out_specs=pl.BlockSpec(memory_space=pltpu.VMEM),
        scratch_shapes=[
            pltpu.VMEM((N_DEV, CHUNK, M), jnp.float32),
            pltpu.VMEM((N_DEV, CHUNK, M), jnp.float32),
            pltpu.SemaphoreType.DMA((N_DEV - 1,)),
            pltpu.SemaphoreType.DMA((N_DEV - 1,)),
            pltpu.SemaphoreType.DMA((N_DEV - 1,)),
            pltpu.SemaphoreType.DMA((N_DEV - 1,)),
            pltpu.SemaphoreType.DMA((N_DEV - 1,)),
            pltpu.SemaphoreType.DMA((N_DEV - 1,)),
            pltpu.SemaphoreType.DMA((N_DEV - 1,)),
            pltpu.SemaphoreType.DMA((N_DEV - 1,)),
        ],
        compiler_params=pltpu.CompilerParams(collective_id=0),
    )(x, w_mat, meta)


# device time: 192501 ns/iter; 2.1282x vs baseline; 1.2485x over previous
import jax
import jax.numpy as jnp
from jax import lax
from jax.experimental import pallas as pl
from jax.experimental.pallas import tpu as pltpu

N_DEV = 16
M = 2048
N_RING = 4
BAND = M // N_RING
CHUNK = BAND // N_DEV
N_HOP = N_DEV - 1

H_RING = [0, 1, 5, 9, 13, 14, 10, 6, 2, 3, 7, 11, 15, 12, 8, 4]
POS = [0] * N_DEV
NEXT = [0] * N_DEV
PREV = [0] * N_DEV
for _p, _l in enumerate(H_RING):
    POS[_l] = _p
    NEXT[_l] = H_RING[(_p + 1) % N_DEV]
    PREV[_l] = H_RING[(_p - 1) % N_DEV]


def kernel(x, w_mat):
    my = lax.axis_index("i")
    meta = jnp.stack(
        [
            jnp.asarray(POS, jnp.int32)[my],
            jnp.asarray(NEXT, jnp.int32)[my],
            jnp.asarray(PREV, jnp.int32)[my],
        ]
    )

    def body(x_ref, w_ref, meta_ref, out_ref, comm,
             rs_send, rs_recv, ag_send, ag_recv):
        pos = meta_ref[0]
        right = meta_ref[1]
        left = meta_ref[2]
        qos = (N_DEV - pos) % N_DEV

        tgt = [right, right, left, left]
        rpos = [pos, pos, qos, qos]
        base = [r * BAND for r in range(N_RING)]

        def local_chunk(r, c):
            return jnp.dot(
                x_ref[pl.ds(base[r] + c * CHUNK, CHUNK), :],
                w_ref[:, :],
                preferred_element_type=jnp.float32,
            )

        def rs_rdma(r, s):
            return pltpu.make_async_remote_copy(
                src_ref=comm.at[r, s],
                dst_ref=comm.at[r, s + 1],
                send_sem=rs_send.at[r, s],
                recv_sem=rs_recv.at[r, s],
                device_id=(tgt[r],),
                device_id_type=pl.DeviceIdType.MESH,
            )

        def ag_rdma(r, s):
            cs = (rpos[r] + 1 - s) % N_DEV
            rows = pl.ds(base[r] + cs * CHUNK, CHUNK)
            return pltpu.make_async_remote_copy(
                src_ref=out_ref.at[rows, :],
                dst_ref=out_ref.at[rows, :],
                send_sem=ag_send.at[r, s],
                recv_sem=ag_recv.at[r, s],
                device_id=(tgt[r],),
                device_id_type=pl.DeviceIdType.MESH,
            )

        for r in range(N_RING):
            comm[r, 0, :, :] = local_chunk(r, rpos[r])

        barrier_sem = pltpu.get_barrier_semaphore()
        pl.semaphore_signal(barrier_sem, 1, device_id=(right,),
                            device_id_type=pl.DeviceIdType.MESH)
        pl.semaphore_signal(barrier_sem, 1, device_id=(left,),
                            device_id_type=pl.DeviceIdType.MESH)
        pl.semaphore_wait(barrier_sem, 2)

        pending = []

        rs = {}
        for r in (0, 2):
            rs[r, 0] = rs_rdma(r, 0)
            rs[r, 0].start()
        for s in range(N_HOP):
            for r in (1, 3):
                rs[r, s] = rs_rdma(r, s)
                rs[r, s].start()
            a0 = local_chunk(0, (pos - s - 1) % N_DEV)
            a2 = local_chunk(2, (qos - s - 1) % N_DEV)
            rs[0, s].wait_recv()
            rs[2, s].wait_recv()
            comm[0, s + 1, :, :] = comm[0, s + 1, :, :] + a0
            comm[2, s + 1, :, :] = comm[2, s + 1, :, :] + a2
            if s < N_HOP - 1:
                for r in (0, 2):
                    rs[r, s + 1] = rs_rdma(r, s + 1)
                    rs[r, s + 1].start()
            a1 = local_chunk(1, (pos - s - 1) % N_DEV)
            a3 = local_chunk(3, (qos - s - 1) % N_DEV)
            rs[1, s].wait_recv()
            rs[3, s].wait_recv()
            comm[1, s + 1, :, :] = comm[1, s + 1, :, :] + a1
            comm[3, s + 1, :, :] = comm[3, s + 1, :, :] + a3
        pending.extend(rs.values())

        for r in range(N_RING):
            own = (rpos[r] + 1) % N_DEV
            out_ref[pl.ds(base[r] + own * CHUNK, CHUNK), :] = jnp.maximum(
                comm[r, N_HOP, :, :], 0.0
            )

        ag = {}
        for r in (0, 2):
            ag[r, 0] = ag_rdma(r, 0)
            ag[r, 0].start()
        for s in range(N_HOP):
            for r in (1, 3):
                ag[r, s] = ag_rdma(r, s)
                ag[r, s].start()
            ag[0, s].wait_recv()
            ag[2, s].wait_recv()
            if s < N_HOP - 1:
                for r in (0, 2):
                    ag[r, s + 1] = ag_rdma(r, s + 1)
                    ag[r, s + 1].start()
            ag[1, s].wait_recv()
            ag[3, s].wait_recv()
        pending.extend(ag.values())

        for d in pending:
            d.wait_send()

    return pl.pallas_call(
        body,
        out_shape=jax.ShapeDtypeStruct((M, M), jnp.float32),
        in_specs=[
            pl.BlockSpec(memory_space=pltpu.VMEM),
            pl.BlockSpec(memory_space=pltpu.VMEM),
            pl.BlockSpec(memory_space=pltpu.SMEM),
        ],
        out_specs=pl.BlockSpec(memory_space=pltpu.VMEM),
        scratch_shapes=[
            pltpu.VMEM((N_RING, N_DEV, CHUNK, M), jnp.float32),
            pltpu.SemaphoreType.DMA((N_RING, N_HOP)),
            pltpu.SemaphoreType.DMA((N_RING, N_HOP)),
            pltpu.SemaphoreType.DMA((N_RING, N_HOP)),
            pltpu.SemaphoreType.DMA((N_RING, N_HOP)),
        ],
        compiler_params=pltpu.CompilerParams(collective_id=0),
    )(x, w_mat, meta)


# device time: 190071 ns/iter; 2.1554x vs baseline; 1.0128x over previous
import jax
import jax.numpy as jnp
from jax import lax
from jax.experimental import pallas as pl
from jax.experimental.pallas import tpu as pltpu

N_DEV = 16
M = 2048
N_RING = 4
BAND = M // N_RING
CHUNK = BAND // N_DEV
N_HOP = N_DEV - 1

H_RING = [0, 1, 5, 9, 13, 14, 10, 6, 2, 3, 7, 11, 15, 12, 8, 4]
POS = [0] * N_DEV
NEXT = [0] * N_DEV
PREV = [0] * N_DEV
for _p, _l in enumerate(H_RING):
    POS[_l] = _p
    NEXT[_l] = H_RING[(_p + 1) % N_DEV]
    PREV[_l] = H_RING[(_p - 1) % N_DEV]


def kernel(x, w_mat):
    my = lax.axis_index("i")
    meta = jnp.stack(
        [
            jnp.asarray(POS, jnp.int32)[my],
            jnp.asarray(NEXT, jnp.int32)[my],
            jnp.asarray(PREV, jnp.int32)[my],
        ]
    )

    def body(x_ref, w_ref, meta_ref, out_ref, comm,
             rs_send, rs_recv, ag_send, ag_recv):
        pos = meta_ref[0]
        right = meta_ref[1]
        left = meta_ref[2]
        qos = (N_DEV - pos) % N_DEV

        tgt = [right, right, left, left]
        rpos = [pos, pos, qos, qos]
        base = [r * BAND for r in range(N_RING)]

        def local_chunk(r, c):
            return jnp.dot(
                x_ref[pl.ds(base[r] + c * CHUNK, CHUNK), :],
                w_ref[:, :],
                preferred_element_type=jnp.float32,
            )

        def rs_rdma(r, s):
            return pltpu.make_async_remote_copy(
                src_ref=comm.at[r, s],
                dst_ref=comm.at[r, s + 1],
                send_sem=rs_send.at[r, s],
                recv_sem=rs_recv.at[r, s],
                device_id=(tgt[r],),
                device_id_type=pl.DeviceIdType.MESH,
            )

        def ag_rdma(r, s):
            cs = (rpos[r] + 1 - s) % N_DEV
            rows = pl.ds(base[r] + cs * CHUNK, CHUNK)
            return pltpu.make_async_remote_copy(
                src_ref=out_ref.at[rows, :],
                dst_ref=out_ref.at[rows, :],
                send_sem=ag_send.at[r, s],
                recv_sem=ag_recv.at[r, s],
                device_id=(tgt[r],),
                device_id_type=pl.DeviceIdType.MESH,
            )

        for r in range(N_RING):
            comm[r, 0, :, :] = local_chunk(r, rpos[r])

        barrier_sem = pltpu.get_barrier_semaphore()
        pl.semaphore_signal(barrier_sem, 1, device_id=(right,),
                            device_id_type=pl.DeviceIdType.MESH)
        pl.semaphore_signal(barrier_sem, 1, device_id=(left,),
                            device_id_type=pl.DeviceIdType.MESH)
        pl.semaphore_wait(barrier_sem, 2)

        pending = []

        def relu_own(r):
            own = (rpos[r] + 1) % N_DEV
            out_ref[pl.ds(base[r] + own * CHUNK, CHUNK), :] = jnp.maximum(
                comm[r, N_HOP, :, :], 0.0
            )

        rs = {}
        ag = {}
        for r in (0, 2):
            rs[r, 0] = rs_rdma(r, 0)
            rs[r, 0].start()
        for s in range(N_HOP):
            last = s == N_HOP - 1
            for r in (1, 3):
                rs[r, s] = rs_rdma(r, s)
                rs[r, s].start()
            a0 = local_chunk(0, (pos - s - 1) % N_DEV)
            a2 = local_chunk(2, (qos - s - 1) % N_DEV)
            rs[0, s].wait_recv()
            comm[0, s + 1, :, :] = comm[0, s + 1, :, :] + a0
            if not last:
                rs[0, s + 1] = rs_rdma(0, s + 1)
                rs[0, s + 1].start()
            else:
                relu_own(0)
                ag[0, 0] = ag_rdma(0, 0)
                ag[0, 0].start()
            rs[2, s].wait_recv()
            comm[2, s + 1, :, :] = comm[2, s + 1, :, :] + a2
            if not last:
                rs[2, s + 1] = rs_rdma(2, s + 1)
                rs[2, s + 1].start()
            else:
                relu_own(2)
                ag[2, 0] = ag_rdma(2, 0)
                ag[2, 0].start()
            a1 = local_chunk(1, (pos - s - 1) % N_DEV)
            a3 = local_chunk(3, (qos - s - 1) % N_DEV)
            rs[1, s].wait_recv()
            comm[1, s + 1, :, :] = comm[1, s + 1, :, :] + a1
            rs[3, s].wait_recv()
            comm[3, s + 1, :, :] = comm[3, s + 1, :, :] + a3
        relu_own(1)
        relu_own(3)
        pending.extend(rs.values())

        for s in range(N_HOP):
            last = s == N_HOP - 1
            for r in (1, 3):
                ag[r, s] = ag_rdma(r, s)
                ag[r, s].start()
            ag[0, s].wait_recv()
            if not last:
                ag[0, s + 1] = ag_rdma(0, s + 1)
                ag[0, s + 1].start()
            ag[2, s].wait_recv()
            if not last:
                ag[2, s + 1] = ag_rdma(2, s + 1)
                ag[2, s + 1].start()
            ag[1, s].wait_recv()
            ag[3, s].wait_recv()
        pending.extend(ag.values())

        for d in pending:
            d.wait_send()

    return pl.pallas_call(
        body,
        out_shape=jax.ShapeDtypeStruct((M, M), jnp.float32),
        in_specs=[
            pl.BlockSpec(memory_space=pltpu.VMEM),
            pl.BlockSpec(memory_space=pltpu.VMEM),
            pl.BlockSpec(memory_space=pltpu.SMEM),
        ],
        out_specs=pl.BlockSpec(memory_space=pltpu.VMEM),
        scratch_shapes=[
            pltpu.VMEM((N_RING, N_DEV, CHUNK, M), jnp.float32),
            pltpu.SemaphoreType.DMA((N_RING, N_HOP)),
            pltpu.SemaphoreType.DMA((N_RING, N_HOP)),
            pltpu.SemaphoreType.DMA((N_RING, N_HOP)),
            pltpu.SemaphoreType.DMA((N_RING, N_HOP)),
        ],
        compiler_params=pltpu.CompilerParams(collective_id=0),
    )(x, w_mat, meta)
